# baseline (speedup 1.0000x reference)
import dataclasses
import functools

import jax
import jax.numpy as jnp
from jax import lax
from jax.experimental import pallas as pl
from jax.experimental.pallas import tpu as pltpu
from jax.experimental.pallas import tpu_sc as plsc

_C = 384
_G = _C // 16
_NW = 32
_TILE = 64


def _sc_shuffle(x_flat, idx, scalar, n_rows):
    rows_per_w = n_rows // _NW
    tiles_per_w = rows_per_w // _TILE
    mesh = plsc.VectorSubcoreMesh(core_axis_name="c", subcore_axis_name="s")
    cp = pltpu.CompilerParams()
    if "needs_layout_passes" in pltpu.CompilerParams.__dataclass_fields__:
        cp = dataclasses.replace(cp, needs_layout_passes=False)

    @functools.partial(
        pl.kernel,
        mesh=mesh,
        compiler_params=cp,
        out_type=jax.ShapeDtypeStruct((n_rows * _C,), jnp.float32),
        scratch_types=[
            pltpu.VMEM((_TILE * _C,), jnp.float32),
            pltpu.VMEM((_TILE * _C,), jnp.float32),
            pltpu.VMEM((_TILE * _C,), jnp.float32),
            pltpu.VMEM((_TILE * _C,), jnp.float32),
            pltpu.VMEM((_C,), jnp.int32),
            pltpu.VMEM((_C,), jnp.float32),
            pltpu.SemaphoreType.DMA,
            pltpu.SemaphoreType.DMA,
            pltpu.SemaphoreType.DMA,
            pltpu.SemaphoreType.DMA,
        ],
    )
    def k(x_hbm, idx_hbm, s_hbm, o_hbm, in0, in1, out0, out1, idx_v, s_v,
          si0, si1, so0, so1):
        cid = lax.axis_index("c")
        sid = lax.axis_index("s")
        wid = sid * 2 + cid
        base = wid * rows_per_w * _C
        pltpu.sync_copy(idx_hbm, idx_v)
        pltpu.sync_copy(s_hbm, s_v)

        def src(t):
            return x_hbm.at[pl.ds(base + t * (_TILE * _C), _TILE * _C)]

        def dst(t):
            return o_hbm.at[pl.ds(base + t * (_TILE * _C), _TILE * _C)]

        def compute(in_v, out_v):
            for g in range(_G):
                cvec = idx_v[pl.ds(g * 16, 16)]
                svec = s_v[pl.ds(g * 16, 16)]

                def body(r, g=g, cvec=cvec, svec=svec, in_v=in_v, out_v=out_v):
                    fvec = cvec + r * _C
                    vals = plsc.load_gather(in_v, [fvec])
                    out_v[pl.ds(r * _C + g * 16, 16)] = vals * svec

                plsc.parallel_loop(0, _TILE, 1, unroll=8)(body)

        pltpu.async_copy(src(0), in0, si0)
        pltpu.async_copy(src(1), in1, si1)

        @pl.loop(0, tiles_per_w, step=2)
        def _(t):
            for leg, in_v, out_v, si, so in (
                (0, in0, out0, si0, so0),
                (1, in1, out1, si1, so1),
            ):
                tt = t + leg
                pltpu.make_async_copy(src(tt), in_v, si).wait()

                @pl.when(tt >= 2)
                def _():
                    pltpu.make_async_copy(out_v, dst(tt), so).wait()

                compute(in_v, out_v)
                pltpu.async_copy(out_v, dst(tt), so)

                @pl.when(tt + 2 < tiles_per_w)
                def _():
                    pltpu.async_copy(src(tt + 2), in_v, si)

        pltpu.make_async_copy(out0, dst(0), so0).wait()
        pltpu.make_async_copy(out1, dst(1), so1).wait()

    return k(x_flat, idx, scalar)


def _tc_body(x_ref, p_ref, s_ref, o_ref):
    x = x_ref[...]
    hi = x.astype(jnp.bfloat16)
    p = p_ref[...]
    acc = jax.lax.dot(hi, p, preferred_element_type=jnp.float32)
    o_ref[...] = acc * s_ref[...]


def _tc_shuffle(x2, shuffle_array, scalar, n, c):
    p = (shuffle_array[None, :] == jnp.arange(c, dtype=jnp.int32)[:, None])
    p = p.astype(jnp.bfloat16)
    s2 = scalar.reshape(1, c)

    block_rows = 7168
    grid = (n // block_rows,)
    return pl.pallas_call(
        _tc_body,
        grid=grid,
        in_specs=[
            pl.BlockSpec((block_rows, c), lambda i: (i, 0)),
            pl.BlockSpec((c, c), lambda i: (0, 0)),
            pl.BlockSpec((1, c), lambda i: (0, 0)),
        ],
        out_specs=pl.BlockSpec((block_rows, c), lambda i: (i, 0)),
        out_shape=jax.ShapeDtypeStruct((n, c), jnp.float32),
    )(x2, p, s2)


def kernel(x, shuffle_array, scalar):
    orig_shape = x.shape
    c = x.shape[-1]
    n = x.size // c
    out = _sc_shuffle(x.reshape(n * c), shuffle_array, scalar, n)
    return out.reshape(orig_shape)


# speedup vs baseline: 2.9291x; 2.9291x over previous
import dataclasses
import functools

import jax
import jax.numpy as jnp
from jax import lax
from jax.experimental import pallas as pl
from jax.experimental.pallas import tpu as pltpu
from jax.experimental.pallas import tpu_sc as plsc

_C = 384
_G = _C // 16
_NW = 32
_TILE = 64


def _sc_shuffle(x2, idx, scalar, n_rows):
    rows_per_w = n_rows // _NW
    tiles_per_w = rows_per_w // _TILE
    mesh = plsc.VectorSubcoreMesh(core_axis_name="c", subcore_axis_name="s")
    cp = pltpu.CompilerParams()
    cp = dataclasses.replace(cp, needs_layout_passes=False,
                             use_tc_tiling_on_sc=True)

    @functools.partial(
        pl.kernel,
        mesh=mesh,
        compiler_params=cp,
        out_type=jax.ShapeDtypeStruct((n_rows, _C), jnp.float32),
        scratch_types=[
            pltpu.VMEM((_TILE, _C), jnp.float32),
            pltpu.VMEM((_TILE, _C), jnp.float32),
            pltpu.VMEM((_TILE, _C), jnp.float32),
            pltpu.VMEM((_TILE, _C), jnp.float32),
            pltpu.VMEM((_C,), jnp.int32),
            pltpu.VMEM((_C,), jnp.float32),
            pltpu.SemaphoreType.DMA,
            pltpu.SemaphoreType.DMA,
            pltpu.SemaphoreType.DMA,
            pltpu.SemaphoreType.DMA,
        ],
    )
    def k(x_hbm, idx_hbm, s_hbm, o_hbm, in0, in1, out0, out1, idx_v, s_v,
          si0, si1, so0, so1):
        cid = lax.axis_index("c")
        sid = lax.axis_index("s")
        wid = sid * 2 + cid
        base = wid * rows_per_w
        pltpu.sync_copy(idx_hbm, idx_v)
        pltpu.sync_copy(s_hbm, s_v)

        def src(t):
            return x_hbm.at[pl.ds(base + t * _TILE, _TILE)]

        def dst(t):
            return o_hbm.at[pl.ds(base + t * _TILE, _TILE)]

        def compute(in_v, out_v):
            zero16 = jnp.zeros((16,), jnp.int32)
            for g in range(_G):
                cvec = idx_v[pl.ds(g * 16, 16)]
                svec = s_v[pl.ds(g * 16, 16)]

                def body(r, g=g, cvec=cvec, svec=svec, in_v=in_v,
                         out_v=out_v, zero16=zero16):
                    rvec = zero16 + r
                    vals = plsc.load_gather(in_v, [rvec, cvec])
                    out_v[r, pl.ds(g * 16, 16)] = vals * svec

                plsc.parallel_loop(0, _TILE, 1, unroll=8)(body)

        pltpu.async_copy(src(0), in0, si0)
        pltpu.async_copy(src(1), in1, si1)

        @pl.loop(0, tiles_per_w, step=2)
        def _(t):
            for leg, in_v, out_v, si, so in (
                (0, in0, out0, si0, so0),
                (1, in1, out1, si1, so1),
            ):
                tt = t + leg
                pltpu.make_async_copy(src(tt), in_v, si).wait()

                @pl.when(tt >= 2)
                def _():
                    pltpu.make_async_copy(out_v, dst(tt), so).wait()

                compute(in_v, out_v)
                pltpu.async_copy(out_v, dst(tt), so)

                @pl.when(tt + 2 < tiles_per_w)
                def _():
                    pltpu.async_copy(src(tt + 2), in_v, si)

        pltpu.make_async_copy(out0, dst(0), so0).wait()
        pltpu.make_async_copy(out1, dst(1), so1).wait()

    return k(x2, idx, scalar)


def _tc_body(x_ref, p_ref, s_ref, o_ref):
    x = x_ref[...]
    hi = x.astype(jnp.bfloat16)
    p = p_ref[...]
    acc = jax.lax.dot(hi, p, preferred_element_type=jnp.float32)
    o_ref[...] = acc * s_ref[...]


def _tc_shuffle(x2, shuffle_array, scalar, n, c):
    p = (shuffle_array[None, :] == jnp.arange(c, dtype=jnp.int32)[:, None])
    p = p.astype(jnp.bfloat16)
    s2 = scalar.reshape(1, c)

    block_rows = 7168
    grid = (n // block_rows,)
    return pl.pallas_call(
        _tc_body,
        grid=grid,
        in_specs=[
            pl.BlockSpec((block_rows, c), lambda i: (i, 0)),
            pl.BlockSpec((c, c), lambda i: (0, 0)),
            pl.BlockSpec((1, c), lambda i: (0, 0)),
        ],
        out_specs=pl.BlockSpec((block_rows, c), lambda i: (i, 0)),
        out_shape=jax.ShapeDtypeStruct((n, c), jnp.float32),
    )(x2, p, s2)


def kernel(x, shuffle_array, scalar):
    orig_shape = x.shape
    c = x.shape[-1]
    n = x.size // c
    out = _sc_shuffle(x.reshape(n, c), shuffle_array, scalar, n)
    return out.reshape(orig_shape)
